# Initial kernel scaffold; baseline (speedup 1.0000x reference)
import jax
import jax.numpy as jnp
from jax.experimental import pallas as pl


def kernel(x, W1, b1, W2, b2, W3, b3, lin1_W, lin1_b, lin2_W, lin2_b, W4, b4, W5, b5, W6, b6):
    raise NotImplementedError("write your pallas kernel here")


# speedup vs baseline: 6.2039x; 6.2039x over previous
import functools

import jax
import jax.numpy as jnp
import numpy as np
from jax.experimental import pallas as pl
from jax.experimental.pallas import tpu as pltpu

F32 = jnp.float32
INF = np.float32(np.inf)


HI = jax.lax.Precision.HIGHEST


def _ec_body(x_ref, xf_ref, xt_ref, wt_ref, b_ref, o_ref, *, N, F, C, BM, SRC):
    i = pl.program_id(0)
    row0 = i * BM

    if SRC is None:
        xb = x_ref[...]
        xfull = xf_ref[...]
        xt = xt_ref[...]
    else:
        hs = x_ref[...]
        hst = xt_ref[...]
        ig = row0 + jax.lax.broadcasted_iota(jnp.int32, (BM, 1), 0)
        js = jax.lax.broadcasted_iota(jnp.int32, (1, SRC), 1)
        oh = jnp.logical_and(js * N <= ig * SRC,
                             ig * SRC < js * N + N).astype(F32)
        xb = jnp.dot(oh, hs, precision=HI, preferred_element_type=F32)
        ia = jax.lax.broadcasted_iota(jnp.int32, (N, 1), 0)
        ja = jax.lax.broadcasted_iota(jnp.int32, (1, SRC), 1)
        ohf = jnp.logical_and(ja * N <= ia * SRC,
                              ia * SRC < ja * N + N).astype(F32)
        xfull = jnp.dot(ohf, hs, precision=HI, preferred_element_type=F32)
        jn = jax.lax.broadcasted_iota(jnp.int32, (1, N), 1)
        si = jax.lax.broadcasted_iota(jnp.int32, (SRC, 1), 0)
        ohT = jnp.logical_and(si * N <= jn * SRC,
                              jn * SRC < si * N + N).astype(F32)
        xt = jnp.dot(hst, ohT, precision=HI, preferred_element_type=F32)

    acc = jnp.zeros((BM, N), F32)
    for f in range(F):
        d = xb[:, f:f + 1] - xt[f:f + 1, :]
        acc = acc + d * d
    rowid = row0 + jax.lax.broadcasted_iota(jnp.int32, (BM, 1), 0)
    colid = jax.lax.broadcasted_iota(jnp.int32, (BM, N), 1)
    acc = acc + jnp.where(rowid == colid, np.float32(1e10), np.float32(0.0))

    xb16 = xb.astype(jnp.bfloat16)
    wt16 = wt_ref[...].astype(jnp.bfloat16)
    out = None
    for _ in range(3):
        m = jnp.min(acc, axis=1, keepdims=True)
        amin = jnp.min(jnp.where(acc == m, colid, N), axis=1, keepdims=True)
        ohk = (colid == amin).astype(F32)
        xj = jnp.dot(ohk, xfull, precision=HI, preferred_element_type=F32)
        feat16 = jnp.concatenate([xb16, (xj - xb).astype(jnp.bfloat16)],
                                 axis=1)
        ok = jnp.dot(feat16, wt16, preferred_element_type=F32)
        out = ok if out is None else jnp.maximum(out, ok)
        acc = jnp.where(colid == amin, INF, acc)
    o_ref[...] = out + b_ref[...]


def _ec(x, W, b, *, N, F, C, BM, SRC=None):
    nb = pl.cdiv(N, BM)
    if SRC is None:
        x_spec = pl.BlockSpec((BM, F), lambda i: (i, 0))
        xf_spec = pl.BlockSpec((N, F), lambda i: (0, 0))
        xt_spec = pl.BlockSpec((F, N), lambda i: (0, 0))
    else:
        x_spec = pl.BlockSpec((SRC, F), lambda i: (0, 0))
        xf_spec = pl.BlockSpec((SRC, F), lambda i: (0, 0))
        xt_spec = pl.BlockSpec((F, SRC), lambda i: (0, 0))
    return pl.pallas_call(
        functools.partial(_ec_body, N=N, F=F, C=C, BM=BM, SRC=SRC),
        grid=(nb,),
        in_specs=[
            x_spec,
            xf_spec,
            xt_spec,
            pl.BlockSpec((2 * F, C), lambda i: (0, 0)),
            pl.BlockSpec((1, C), lambda i: (0, 0)),
        ],
        out_specs=pl.BlockSpec((BM, C), lambda i: (i, 0)),
        out_shape=jax.ShapeDtypeStruct((N, C), F32),
    )(x, x, x.T, W.T, b.reshape(1, C))


def _fps_body(hb_ref, h_ref, ht_ref, o_ref, d_ref, *, N, F, NOUT, BM, NB):
    i = pl.program_id(0)

    @pl.when(i < NB)
    def _build():
        xb = hb_ref[...]
        acc = jnp.zeros((BM, N), F32)
        for f in range(F):
            d = xb[:, f:f + 1] - ht_ref[f:f + 1, :]
            acc = acc + d * d
        d_ref[pl.ds(i * BM, BM), :] = acc

    @pl.when(i == NB)
    def _select():
        colid = jax.lax.broadcasted_iota(jnp.int32, (1, N), 1)
        sidx = jax.lax.broadcasted_iota(jnp.int32, (NOUT, 1), 0)

        def body(k, st):
            dists, idxs, prev = st
            drow = d_ref[pl.ds(prev, 1), :]
            dists = jnp.minimum(dists, drow)
            m = jnp.max(dists, axis=1, keepdims=True)
            cand = jnp.where(dists == m, colid, N)
            nxt = jnp.min(cand)
            idxs = jnp.where(sidx == k, nxt, idxs)
            return dists, idxs, nxt

        dists0 = jnp.full((1, N), INF, F32)
        idxs0 = jnp.zeros((NOUT, 1), jnp.int32)
        _, idxs, _ = jax.lax.fori_loop(1, NOUT, body,
                                       (dists0, idxs0, np.int32(0)))
        oh = (idxs == colid).astype(F32)
        o_ref[...] = jnp.dot(oh, h_ref[...], precision=HI,
                             preferred_element_type=F32)


def _fps(h, *, N, F, NOUT, BM):
    NB = pl.cdiv(N, BM)
    return pl.pallas_call(
        functools.partial(_fps_body, N=N, F=F, NOUT=NOUT, BM=BM, NB=NB),
        grid=(NB + 1,),
        in_specs=[
            pl.BlockSpec((BM, F), lambda i: (jnp.minimum(i, NB - 1), 0)),
            pl.BlockSpec((N, F), lambda i: (0, 0)),
            pl.BlockSpec((F, N), lambda i: (0, 0)),
        ],
        out_specs=pl.BlockSpec((NOUT, F), lambda i: (0, 0)),
        out_shape=jax.ShapeDtypeStruct((NOUT, F), F32),
        scratch_shapes=[pltpu.VMEM((NB * BM, N), F32)],
    )(h, h, h.T)


def _mlp_body(z_ref, w1t_ref, b1_ref, w2t_ref, b2_ref, o_ref):
    z16 = z_ref[...].astype(jnp.bfloat16)
    w1t16 = w1t_ref[...].astype(jnp.bfloat16)
    z1 = jnp.dot(z16, w1t16, preferred_element_type=F32) + b1_ref[...]
    z116 = z1.astype(jnp.bfloat16)
    w2t16 = w2t_ref[...].astype(jnp.bfloat16)
    o_ref[...] = jnp.dot(z116, w2t16, preferred_element_type=F32) + b2_ref[...]


def _mlp(z, lin1_W, lin1_b, lin2_W, lin2_b):
    return pl.pallas_call(
        _mlp_body,
        grid=(1,),
        in_specs=[
            pl.BlockSpec((1, 600), lambda i: (0, 0)),
            pl.BlockSpec((600, 30), lambda i: (0, 0)),
            pl.BlockSpec((1, 30), lambda i: (0, 0)),
            pl.BlockSpec((30, 600), lambda i: (0, 0)),
            pl.BlockSpec((1, 600), lambda i: (0, 0)),
        ],
        out_specs=pl.BlockSpec((1, 600), lambda i: (0, 0)),
        out_shape=jax.ShapeDtypeStruct((1, 600), F32),
    )(z, lin1_W.T, lin1_b.reshape(1, 30), lin2_W.T, lin2_b.reshape(1, 600))


def kernel(x, W1, b1, W2, b2, W3, b3, lin1_W, lin1_b, lin2_W, lin2_b,
           W4, b4, W5, b5, W6, b6):
    h1 = _ec(x, W1, b1, N=2300, F=3, C=9, BM=256)
    h1s = _fps(h1, N=2300, F=9, NOUT=500, BM=256)
    h2 = _ec(h1s, W2, b2, N=500, F=9, C=12, BM=512)
    h2s = _fps(h2, N=500, F=12, NOUT=50, BM=512)
    h3 = _ec(h2s, W3, b3, N=50, F=12, C=12, BM=64)
    z = _mlp(h3.reshape(1, 600), lin1_W, lin1_b, lin2_W, lin2_b)
    h4 = _ec(z.reshape(50, 12), W4, b4, N=50, F=12, C=12, BM=64)
    h5 = _ec(h4, W5, b5, N=500, F=12, C=9, BM=512, SRC=50)
    out = _ec(h5, W6, b6, N=2300, F=9, C=3, BM=256, SRC=500)
    return out


# speedup vs baseline: 9.3026x; 1.4995x over previous
import functools

import jax
import jax.numpy as jnp
import numpy as np
from jax.experimental import pallas as pl
from jax.experimental.pallas import tpu as pltpu

F32 = jnp.float32
INF = np.float32(np.inf)


def _split3(x):
    x1 = x.astype(jnp.bfloat16)
    r1 = x - x1.astype(F32)
    x2 = r1.astype(jnp.bfloat16)
    x3 = (r1 - x2.astype(F32)).astype(jnp.bfloat16)
    return x1, x2, x3


def _gdot(oh, x):
    oh16 = oh.astype(jnp.bfloat16)
    x1, x2, x3 = _split3(x)
    a = jnp.dot(oh16, x1, preferred_element_type=F32)
    b = jnp.dot(oh16, x2, preferred_element_type=F32)
    c = jnp.dot(oh16, x3, preferred_element_type=F32)
    return (a + b) + c


def _gdotr(x, oh):
    oh16 = oh.astype(jnp.bfloat16)
    x1, x2, x3 = _split3(x)
    a = jnp.dot(x1, oh16, preferred_element_type=F32)
    b = jnp.dot(x2, oh16, preferred_element_type=F32)
    c = jnp.dot(x3, oh16, preferred_element_type=F32)
    return (a + b) + c


def _ec_body(x_ref, xf_ref, xt_ref, wt_ref, b_ref, o_ref, *scr, N, F, C, BM,
             SRC):
    i = pl.program_id(0)
    row0 = i * BM

    if SRC is None:
        xb = x_ref[...]
        xfull = xf_ref[...]
        xt = xt_ref[...]
    else:
        xfs_ref, xts_ref = scr

        @pl.when(i == 0)
        def _expand():
            hs = x_ref[...]
            hst = xt_ref[...]
            ia = jax.lax.broadcasted_iota(jnp.int32, (N, 1), 0)
            ja = jax.lax.broadcasted_iota(jnp.int32, (1, SRC), 1)
            ohf = jnp.logical_and(ja * N <= ia * SRC,
                                  ia * SRC < ja * N + N).astype(F32)
            xfs_ref[0:N, :] = _gdot(ohf, hs)
            jn = jax.lax.broadcasted_iota(jnp.int32, (1, N), 1)
            si = jax.lax.broadcasted_iota(jnp.int32, (SRC, 1), 0)
            ohT = jnp.logical_and(si * N <= jn * SRC,
                                  jn * SRC < si * N + N).astype(F32)
            xts_ref[...] = _gdotr(hst, ohT)

        xb = xfs_ref[pl.ds(row0, BM), :]
        xfull = xfs_ref[0:N, :]
        xt = xts_ref[...]

    acc = jnp.zeros((BM, N), F32)
    for f in range(F):
        d = xb[:, f:f + 1] - xt[f:f + 1, :]
        acc = acc + d * d
    rowid = row0 + jax.lax.broadcasted_iota(jnp.int32, (BM, 1), 0)
    colid = jax.lax.broadcasted_iota(jnp.int32, (BM, N), 1)
    acc = acc + jnp.where(rowid == colid, np.float32(1e10), np.float32(0.0))

    xb16 = xb.astype(jnp.bfloat16)
    wt16 = wt_ref[...].astype(jnp.bfloat16)
    out = None
    for _ in range(3):
        m = jnp.min(acc, axis=1, keepdims=True)
        amin = jnp.min(jnp.where(acc == m, colid, N), axis=1, keepdims=True)
        ohk = (colid == amin).astype(F32)
        xj = _gdot(ohk, xfull)
        feat16 = jnp.concatenate([xb16, (xj - xb).astype(jnp.bfloat16)],
                                 axis=1)
        ok = jnp.dot(feat16, wt16, preferred_element_type=F32)
        out = ok if out is None else jnp.maximum(out, ok)
        acc = jnp.where(colid == amin, INF, acc)
    o_ref[...] = out + b_ref[...]


def _ec(x, W, b, *, N, F, C, BM, SRC=None):
    nb = pl.cdiv(N, BM)
    if SRC is None:
        x_spec = pl.BlockSpec((BM, F), lambda i: (i, 0))
        xf_spec = pl.BlockSpec((N, F), lambda i: (0, 0))
        xt_spec = pl.BlockSpec((F, N), lambda i: (0, 0))
    else:
        x_spec = pl.BlockSpec((SRC, F), lambda i: (0, 0))
        xf_spec = pl.BlockSpec((SRC, F), lambda i: (0, 0))
        xt_spec = pl.BlockSpec((F, SRC), lambda i: (0, 0))
    scratch = []
    if SRC is not None:
        scratch = [pltpu.VMEM((nb * BM, F), F32), pltpu.VMEM((F, N), F32)]
    return pl.pallas_call(
        functools.partial(_ec_body, N=N, F=F, C=C, BM=BM, SRC=SRC),
        grid=(nb,),
        in_specs=[
            x_spec,
            xf_spec,
            xt_spec,
            pl.BlockSpec((2 * F, C), lambda i: (0, 0)),
            pl.BlockSpec((1, C), lambda i: (0, 0)),
        ],
        out_specs=pl.BlockSpec((BM, C), lambda i: (i, 0)),
        out_shape=jax.ShapeDtypeStruct((N, C), F32),
        scratch_shapes=scratch,
    )(x, x, x.T, W.T, b.reshape(1, C))


def _fps_body(hb_ref, h_ref, ht_ref, o_ref, d_ref, *, N, F, NOUT, BM, NB,
              LOOPN):
    i = pl.program_id(0)

    @pl.when(i < NB)
    def _build():
        xb = hb_ref[...]
        acc = jnp.zeros((BM, N), F32)
        for f in range(F):
            d = xb[:, f:f + 1] - ht_ref[f:f + 1, :]
            acc = acc + d * d
        d_ref[pl.ds(i * BM, BM), :] = acc

    @pl.when(i == NB)
    def _select():
        colid = jax.lax.broadcasted_iota(jnp.int32, (1, N), 1)
        sidx = jax.lax.broadcasted_iota(jnp.int32, (NOUT, 1), 0)

        def body(k, st):
            dists, idxs, prev = st
            drow = d_ref[pl.ds(prev, 1), :]
            dists = jnp.minimum(dists, drow)
            m = jnp.max(dists, axis=1, keepdims=True)
            cand = jnp.where(dists == m, colid, N)
            nxt = jnp.min(cand)
            idxs = jnp.where(sidx == k, nxt, idxs)
            return dists, idxs, nxt

        dists0 = jnp.full((1, N), INF, F32)
        idxs0 = jnp.zeros((NOUT, 1), jnp.int32)
        _, idxs, _ = jax.lax.fori_loop(1, LOOPN, body,
                                       (dists0, idxs0, np.int32(0)))
        oh = (idxs == colid).astype(F32)
        o_ref[...] = _gdot(oh, h_ref[...])


def _fps(h, *, N, F, NOUT, BM, PROBE=None):
    NB = pl.cdiv(N, BM)
    return pl.pallas_call(
        functools.partial(_fps_body, N=N, F=F, NOUT=NOUT, BM=BM, NB=NB,
                          LOOPN=(PROBE or NOUT)),
        grid=(NB + 1,),
        in_specs=[
            pl.BlockSpec((BM, F), lambda i: (jnp.minimum(i, NB - 1), 0)),
            pl.BlockSpec((N, F), lambda i: (0, 0)),
            pl.BlockSpec((F, N), lambda i: (0, 0)),
        ],
        out_specs=pl.BlockSpec((NOUT, F), lambda i: (0, 0)),
        out_shape=jax.ShapeDtypeStruct((NOUT, F), F32),
        scratch_shapes=[pltpu.VMEM((NB * BM, N), F32)],
    )(h, h, h.T)


def _mlp_body(z_ref, w1t_ref, b1_ref, w2t_ref, b2_ref, o_ref):
    z16 = z_ref[...].astype(jnp.bfloat16)
    w1t16 = w1t_ref[...].astype(jnp.bfloat16)
    z1 = jnp.dot(z16, w1t16, preferred_element_type=F32) + b1_ref[...]
    z116 = z1.astype(jnp.bfloat16)
    w2t16 = w2t_ref[...].astype(jnp.bfloat16)
    o_ref[...] = jnp.dot(z116, w2t16, preferred_element_type=F32) + b2_ref[...]


def _mlp(z, lin1_W, lin1_b, lin2_W, lin2_b):
    return pl.pallas_call(
        _mlp_body,
        grid=(1,),
        in_specs=[
            pl.BlockSpec((1, 600), lambda i: (0, 0)),
            pl.BlockSpec((600, 30), lambda i: (0, 0)),
            pl.BlockSpec((1, 30), lambda i: (0, 0)),
            pl.BlockSpec((30, 600), lambda i: (0, 0)),
            pl.BlockSpec((1, 600), lambda i: (0, 0)),
        ],
        out_specs=pl.BlockSpec((1, 600), lambda i: (0, 0)),
        out_shape=jax.ShapeDtypeStruct((1, 600), F32),
    )(z, lin1_W.T, lin1_b.reshape(1, 30), lin2_W.T, lin2_b.reshape(1, 600))


def kernel(x, W1, b1, W2, b2, W3, b3, lin1_W, lin1_b, lin2_W, lin2_b,
           W4, b4, W5, b5, W6, b6):
    h1 = _ec(x, W1, b1, N=2300, F=3, C=9, BM=256)
    h1s = _fps(h1, N=2300, F=9, NOUT=500, BM=256)
    h2 = _ec(h1s, W2, b2, N=500, F=9, C=12, BM=512)
    h2s = _fps(h2, N=500, F=12, NOUT=50, BM=512)
    h3 = _ec(h2s, W3, b3, N=50, F=12, C=12, BM=64)
    z = _mlp(h3.reshape(1, 600), lin1_W, lin1_b, lin2_W, lin2_b)
    h4 = _ec(z.reshape(50, 12), W4, b4, N=50, F=12, C=12, BM=64)
    h5 = _ec(h4, W5, b5, N=500, F=12, C=9, BM=512, SRC=50)
    out = _ec(h5, W6, b6, N=2300, F=9, C=3, BM=256, SRC=500)
    return out
